# Initial kernel scaffold; baseline (speedup 1.0000x reference)
import jax
import jax.numpy as jnp
from jax.experimental import pallas as pl


def kernel(reg, targets):
    raise NotImplementedError("write your pallas kernel here")


# speedup vs baseline: 1.9140x; 1.9140x over previous
import jax
import jax.numpy as jnp
from jax.experimental import pallas as pl
from jax.experimental.pallas import tpu as pltpu

_ROWS = 32 * 512
_COLS = 512
_BLOCK_ROWS = 2048
_GRID = _ROWS // _BLOCK_ROWS
_TOTAL = float(_ROWS * _COLS)


def _body(reg_ref, tgt_ref, out_ref, acc_ref):
    i = pl.program_id(0)
    r = reg_ref[...]
    t = tgt_ref[...]
    d = r - t
    d2 = d * d
    mf = t > 0.0
    fsum = jnp.sum(jnp.where(mf, d2, 0.0))
    usum = jnp.sum(jnp.where(mf, 0.0, d2))
    fcnt = jnp.sum(jnp.where(mf, 1.0, 0.0))

    @pl.when(i == 0)
    def _():
        acc_ref[0] = fsum
        acc_ref[1] = usum
        acc_ref[2] = fcnt

    @pl.when(i > 0)
    def _():
        acc_ref[0] += fsum
        acc_ref[1] += usum
        acc_ref[2] += fcnt

    @pl.when(i == _GRID - 1)
    def _():
        sf = acc_ref[0]
        su = acc_ref[1]
        nf = acc_ref[2]
        nu = _TOTAL - nf
        flood = jnp.where(nf > 0.0, sf / jnp.maximum(nf, 1.0), 0.0)
        unflood = jnp.where(nu > 0.0, su / jnp.maximum(nu, 1.0), 0.0)
        loss_reg = 20.0 * flood + unflood
        out_ref[0] = 2.0 * loss_reg
        out_ref[1] = 2.0 * loss_reg
        out_ref[2] = 2.0 * flood
        out_ref[3] = 2.0 * unflood
        out_ref[4] = loss_reg
        out_ref[5] = flood
        out_ref[6] = unflood
        out_ref[7] = 0.0


@jax.jit
def _run(reg, targets):
    reg2 = reg.reshape(_ROWS, _COLS)
    tgt2 = targets.reshape(_ROWS, _COLS)
    out = pl.pallas_call(
        _body,
        grid=(_GRID,),
        in_specs=[
            pl.BlockSpec((_BLOCK_ROWS, _COLS), lambda i: (i, 0)),
            pl.BlockSpec((_BLOCK_ROWS, _COLS), lambda i: (i, 0)),
        ],
        out_specs=pl.BlockSpec(memory_space=pltpu.SMEM),
        out_shape=jax.ShapeDtypeStruct((8,), jnp.float32),
        scratch_shapes=[pltpu.SMEM((4,), jnp.float32)],
        compiler_params=pltpu.CompilerParams(
            dimension_semantics=("arbitrary",)
        ),
    )(reg2, tgt2)
    return (
        out[0:1],
        out[1],
        out[2],
        out[3],
        out[4],
        out[5],
        out[6],
        out[7:8],
    )


def kernel(reg, targets):
    return _run(reg, targets)
